# baseline (speedup 1.0000x reference)
import functools

import jax
import jax.numpy as jnp
from jax import lax
from jax.experimental import pallas as pl
from jax.experimental.pallas import tpu as pltpu
from jax.experimental.pallas import tpu_sc as plsc

_F32 = jnp.float32
_LN2 = 0.6931471805599453
_SQRT2 = 1.4142135623730951

_S_OFF = 0
_A_OFF = 12
_B_OFF = 44
_C_OFF = 76


def _vlog(x):
    bits = lax.bitcast_convert_type(x, jnp.int32)
    e = lax.shift_right_arithmetic(bits, 23) - 127
    m = lax.bitcast_convert_type(
        jnp.bitwise_or(jnp.bitwise_and(bits, 0x007FFFFF), 0x3F800000), _F32)
    adj = m > _SQRT2
    m = jnp.where(adj, m * 0.5, m)
    e = (e + jnp.where(adj, 1, 0)).astype(_F32)
    t = (m - 1.0) / (m + 1.0)
    t2 = t * t
    p = jnp.float32(1.0 / 9.0)
    p = p * t2 + 1.0 / 7.0
    p = p * t2 + 1.0 / 5.0
    p = p * t2 + 1.0 / 3.0
    p = p * t2 + 1.0
    return 2.0 * t * p + e * _LN2


def _body(yp_hbm, yt_hbm, probs_hbm, d_hbm, yp_v, yt_v, out_v, sem1, sem2):
    if True:
        cp1 = pltpu.async_copy(yp_hbm, yp_v, sem1)
        cp2 = pltpu.async_copy(yt_hbm, yt_v, sem2)
        cp1.wait()
        cp2.wait()

        lane = lax.iota(jnp.int32, 16)
        a = lax.shift_right_logical(lane, 2)
        c = jnp.bitwise_and(lane, 3)
        a8 = a * 8
        c2 = c * 2

        s0a = plsc.load_gather(yp_v, [a + _S_OFF])
        s2c = plsc.load_gather(yp_v, [c + (_S_OFF + 8)])
        cw = [s2c * plsc.load_gather(yp_v, [a8 + c2 + (_C_OFF + o2)])
              for o2 in (0, 1)]

        m = [[None, None], [None, None]]
        for b in range(4):
            s1b = plsc.load_gather(yp_v, [lane * 0 + (_S_OFF + 4 + b)])
            f = s0a * s1b
            ag = [plsc.load_gather(yp_v, [a8 + (_A_OFF + 2 * b + o0)])
                  for o0 in (0, 1)]
            bg = [plsc.load_gather(yp_v, [c2 + (_B_OFF + 8 * b + o1)])
                  for o1 in (0, 1)]
            for o0 in (0, 1):
                t0 = f * ag[o0]
                for o1 in (0, 1):
                    term = t0 * bg[o1]
                    m[o0][o1] = term if b == 0 else m[o0][o1] + term

        probs = s0a * 0.0
        for o0 in (0, 1):
            for o1 in (0, 1):
                for o2 in (0, 1):
                    val = jnp.sum(m[o0][o1] * cw[o2])
                    probs = jnp.where(lane == (o0 * 4 + o1 * 2 + o2),
                                      val, probs)

        yt = jnp.where(lane < 8,
                       plsc.load_gather(yt_v, [jnp.bitwise_and(lane, 7)]),
                       0.0)
        pc = jnp.minimum(jnp.maximum(probs, 1e-10), 1.0)
        d = jnp.sum(yt * (_vlog(yt + 1e-10) - _vlog(pc)))

        out_v[...] = jnp.where(lane == 8, d, probs)
        pltpu.sync_copy(out_v.at[pl.ds(0, 8)], probs_hbm)
        pltpu.sync_copy(out_v.at[pl.ds(8, 1)], d_hbm)


_loss_call = functools.partial(
    pl.kernel,
    mesh=plsc.VectorSubcoreMesh(core_axis_name="c", subcore_axis_name="s",
                                num_cores=1, num_subcores=1),
    compiler_params=pltpu.CompilerParams(needs_layout_passes=False),
    out_type=[
        jax.ShapeDtypeStruct((8,), _F32),
        jax.ShapeDtypeStruct((1,), _F32),
    ],
    scratch_types=[
        pltpu.VMEM((108,), _F32),
        pltpu.VMEM((8,), _F32),
        pltpu.VMEM((16,), _F32),
        pltpu.SemaphoreType.DMA,
        pltpu.SemaphoreType.DMA,
    ],
)(_body)


def kernel(y_pred, y_true):
    probs, d1 = _loss_call(jnp.ravel(y_pred), y_true)
    return (jnp.reshape(d1, ()), probs)


# speedup vs baseline: 1.0049x; 1.0049x over previous
import functools

import jax
import jax.numpy as jnp
from jax import lax
from jax.experimental import pallas as pl
from jax.experimental.pallas import tpu as pltpu
from jax.experimental.pallas import tpu_sc as plsc

_F32 = jnp.float32
_LN2 = 0.6931471805599453
_SQRT2 = 1.4142135623730951

_S_OFF = 0
_A_OFF = 12
_B_OFF = 44
_C_OFF = 76


def _vlog(x):
    bits = lax.bitcast_convert_type(x, jnp.int32)
    e = lax.shift_right_arithmetic(bits, 23) - 127
    m = lax.bitcast_convert_type(
        jnp.bitwise_or(jnp.bitwise_and(bits, 0x007FFFFF), 0x3F800000), _F32)
    adj = m > _SQRT2
    m = jnp.where(adj, m * 0.5, m)
    e = (e + jnp.where(adj, 1, 0)).astype(_F32)
    t = (m - 1.0) / (m + 1.0)
    t2 = t * t
    p = jnp.float32(1.0 / 9.0)
    p = p * t2 + 1.0 / 7.0
    p = p * t2 + 1.0 / 5.0
    p = p * t2 + 1.0 / 3.0
    p = p * t2 + 1.0
    return 2.0 * t * p + e * _LN2


def _body(yp_hbm, yt_hbm, probs_hbm, d_hbm, yp_v, yt_v, out_v, sem1, sem2):
    if True:
        cp1 = pltpu.async_copy(yp_hbm, yp_v, sem1)
        cp2 = pltpu.async_copy(yt_hbm, yt_v, sem2)
        cp1.wait()
        cp2.wait()

        lane = lax.iota(jnp.int32, 16)
        a = lax.shift_right_logical(lane, 2)
        c = jnp.bitwise_and(lane, 3)
        a8 = a * 8
        c2 = c * 2

        s0a = plsc.load_gather(yp_v, [a + _S_OFF])
        s2c = plsc.load_gather(yp_v, [c + (_S_OFF + 8)])
        cw = [s2c * plsc.load_gather(yp_v, [a8 + c2 + (_C_OFF + o2)])
              for o2 in (0, 1)]

        m = [[None, None], [None, None]]
        for b in range(4):
            s1b = plsc.load_gather(yp_v, [lane * 0 + (_S_OFF + 4 + b)])
            f = s0a * s1b
            ag = [plsc.load_gather(yp_v, [a8 + (_A_OFF + 2 * b + o0)])
                  for o0 in (0, 1)]
            bg = [plsc.load_gather(yp_v, [c2 + (_B_OFF + 8 * b + o1)])
                  for o1 in (0, 1)]
            for o0 in (0, 1):
                t0 = f * ag[o0]
                for o1 in (0, 1):
                    term = t0 * bg[o1]
                    m[o0][o1] = term if b == 0 else m[o0][o1] + term

        probs = s0a * 0.0
        for o0 in (0, 1):
            for o1 in (0, 1):
                for o2 in (0, 1):
                    val = jnp.sum(m[o0][o1] * cw[o2])
                    probs = jnp.where(lane == (o0 * 4 + o1 * 2 + o2),
                                      val, probs)

        yt = jnp.where(lane < 8,
                       plsc.load_gather(yt_v, [jnp.bitwise_and(lane, 7)]),
                       0.0)
        pc = jnp.minimum(jnp.maximum(probs, 1e-10), 1.0)
        d = jnp.sum(yt * (_vlog(yt + 1e-10) - _vlog(pc)))

        out_v[...] = jnp.where(lane == 8, d, probs)
        cpo1 = pltpu.async_copy(out_v.at[pl.ds(0, 8)], probs_hbm, sem1)
        cpo2 = pltpu.async_copy(out_v.at[pl.ds(8, 1)], d_hbm, sem2)
        cpo1.wait()
        cpo2.wait()


_loss_call = functools.partial(
    pl.kernel,
    mesh=plsc.VectorSubcoreMesh(core_axis_name="c", subcore_axis_name="s",
                                num_cores=1, num_subcores=1),
    compiler_params=pltpu.CompilerParams(needs_layout_passes=False),
    out_type=[
        jax.ShapeDtypeStruct((8,), _F32),
        jax.ShapeDtypeStruct((1,), _F32),
    ],
    scratch_types=[
        pltpu.VMEM((108,), _F32),
        pltpu.VMEM((8,), _F32),
        pltpu.VMEM((16,), _F32),
        pltpu.SemaphoreType.DMA,
        pltpu.SemaphoreType.DMA,
    ],
)(_body)


def kernel(y_pred, y_true):
    probs, d1 = _loss_call(jnp.ravel(y_pred), y_true)
    return (jnp.reshape(d1, ()), probs)


# speedup vs baseline: 1.0055x; 1.0006x over previous
import functools

import jax
import jax.numpy as jnp
from jax import lax
from jax.experimental import pallas as pl
from jax.experimental.pallas import tpu as pltpu
from jax.experimental.pallas import tpu_sc as plsc

_F32 = jnp.float32
_LN2 = 0.6931471805599453
_SQRT2 = 1.4142135623730951

_S_OFF = 0
_A_OFF = 12
_B_OFF = 44
_C_OFF = 76


def _vlog(x):
    bits = lax.bitcast_convert_type(x, jnp.int32)
    e = lax.shift_right_arithmetic(bits, 23) - 127
    m = lax.bitcast_convert_type(
        jnp.bitwise_or(jnp.bitwise_and(bits, 0x007FFFFF), 0x3F800000), _F32)
    adj = m > _SQRT2
    m = jnp.where(adj, m * 0.5, m)
    e = (e + jnp.where(adj, 1, 0)).astype(_F32)
    t = (m - 1.0) / (m + 1.0)
    t2 = t * t
    p = jnp.float32(1.0 / 9.0)
    p = p * t2 + 1.0 / 7.0
    p = p * t2 + 1.0 / 5.0
    p = p * t2 + 1.0 / 3.0
    p = p * t2 + 1.0
    return 2.0 * t * p + e * _LN2


def _body(yp_hbm, yt_hbm, probs_hbm, d_hbm, yp_v, yt_v, out_v, d_v,
          sem1, sem2):
    if True:
        cp1 = pltpu.async_copy(yp_hbm, yp_v, sem1)
        cp2 = pltpu.async_copy(yt_hbm, yt_v, sem2)
        cp1.wait()
        cp2.wait()

        lane = lax.iota(jnp.int32, 16)
        a = lax.shift_right_logical(lane, 2)
        c = jnp.bitwise_and(lane, 3)
        a8 = a * 8
        c2 = c * 2

        s0a = plsc.load_gather(yp_v, [a + _S_OFF])
        s2c = plsc.load_gather(yp_v, [c + (_S_OFF + 8)])
        cw = [s2c * plsc.load_gather(yp_v, [a8 + c2 + (_C_OFF + o2)])
              for o2 in (0, 1)]

        m = [[None, None], [None, None]]
        for b in range(4):
            s1b = plsc.load_gather(yp_v, [lane * 0 + (_S_OFF + 4 + b)])
            f = s0a * s1b
            ag = [plsc.load_gather(yp_v, [a8 + (_A_OFF + 2 * b + o0)])
                  for o0 in (0, 1)]
            bg = [plsc.load_gather(yp_v, [c2 + (_B_OFF + 8 * b + o1)])
                  for o1 in (0, 1)]
            for o0 in (0, 1):
                t0 = f * ag[o0]
                for o1 in (0, 1):
                    term = t0 * bg[o1]
                    m[o0][o1] = term if b == 0 else m[o0][o1] + term

        probs = s0a * 0.0
        for o0 in (0, 1):
            for o1 in (0, 1):
                for o2 in (0, 1):
                    val = jnp.sum(m[o0][o1] * cw[o2])
                    probs = jnp.where(lane == (o0 * 4 + o1 * 2 + o2),
                                      val, probs)

        out_v[...] = probs
        cpo1 = pltpu.async_copy(out_v.at[pl.ds(0, 8)], probs_hbm, sem1)

        yt = jnp.where(lane < 8,
                       plsc.load_gather(yt_v, [jnp.bitwise_and(lane, 7)]),
                       0.0)
        pc = jnp.minimum(jnp.maximum(probs, 1e-10), 1.0)
        d = jnp.sum(yt * (_vlog(yt + 1e-10) - _vlog(pc)))

        d_v[...] = probs * 0.0 + d
        cpo2 = pltpu.async_copy(d_v.at[pl.ds(0, 1)], d_hbm, sem2)
        cpo1.wait()
        cpo2.wait()


_loss_call = functools.partial(
    pl.kernel,
    mesh=plsc.VectorSubcoreMesh(core_axis_name="c", subcore_axis_name="s",
                                num_cores=1, num_subcores=1),
    compiler_params=pltpu.CompilerParams(needs_layout_passes=False),
    out_type=[
        jax.ShapeDtypeStruct((8,), _F32),
        jax.ShapeDtypeStruct((1,), _F32),
    ],
    scratch_types=[
        pltpu.VMEM((108,), _F32),
        pltpu.VMEM((8,), _F32),
        pltpu.VMEM((16,), _F32),
        pltpu.VMEM((16,), _F32),
        pltpu.SemaphoreType.DMA,
        pltpu.SemaphoreType.DMA,
    ],
)(_body)


def kernel(y_pred, y_true):
    probs, d1 = _loss_call(jnp.ravel(y_pred), y_true)
    return (jnp.reshape(d1, ()), probs)
